# baseline (speedup 1.0000x reference)
import functools

import jax
import jax.numpy as jnp
from jax import lax
from jax.experimental import pallas as pl
from jax.experimental.pallas import tpu as pltpu
from jax.experimental.pallas import tpu_sc as plsc

B, T, D = 1, 2048, 1024
H, DH = 16, 64
E, K, FF = 8, 2, 4096
EPS = 1e-5

TB = 256
NT = T // TB
BS = 512
NB = 15
RPAD = NB * BS
FB = 1024
JB = FF // FB

NEG = float("-inf")
f32 = jnp.float32
bf16 = jnp.bfloat16
i32 = jnp.int32


def _bdot(a, b):
    return jax.lax.dot(a.astype(bf16), b.astype(bf16),
                       preferred_element_type=f32)


def _ln(xb, g, b):
    m = jnp.mean(xb, axis=1, keepdims=True)
    c = xb - m
    v = jnp.mean(c * c, axis=1, keepdims=True)
    return g * c / jnp.sqrt(v + EPS) + b


def _k1_body(x_ref, w_ref, g_ref, b_ref, cos_ref, sin_ref, o_ref):
    h = _ln(x_ref[...], g_ref[...], b_ref[...])
    qkv = jax.lax.dot(h.astype(bf16), w_ref[...], preferred_element_type=f32)
    lane = jax.lax.broadcasted_iota(i32, qkv.shape, 1)
    r_left = jnp.roll(qkv, -1, axis=1)
    r_right = jnp.roll(qkv, 1, axis=1)
    swapped = jnp.where(lane % 2 == 0, r_left, r_right)
    o_ref[...] = qkv * cos_ref[...] + swapped * sin_ref[...]


def _k1(x2d, wqkv_bf, g1, b1, cosT, sinT):
    return pl.pallas_call(
        _k1_body,
        grid=(NT,),
        in_specs=[
            pl.BlockSpec((TB, D), lambda i: (i, 0)),
            pl.BlockSpec((D, 3 * D), lambda i: (0, 0)),
            pl.BlockSpec((1, D), lambda i: (0, 0)),
            pl.BlockSpec((1, D), lambda i: (0, 0)),
            pl.BlockSpec((TB, 3 * D), lambda i: (i, 0)),
            pl.BlockSpec((TB, 3 * D), lambda i: (i, 0)),
        ],
        out_specs=pl.BlockSpec((TB, 3 * D), lambda i: (i, 0)),
        out_shape=jax.ShapeDtypeStruct((T, 3 * D), f32),
    )(x2d, wqkv_bf, g1, b1, cosT, sinT)


KVB = 512
NKV = T // KVB


def _k2_body(q_ref, k_ref, v_ref, o_ref, s_scr, mx_scr, ls_scr, acc_scr):
    qi = pl.program_id(1)
    q_end = qi * TB + TB
    acc_scr[...] = jnp.zeros_like(acc_scr)
    for hh in range(2):
        sl = slice(hh * DH, (hh + 1) * DH)
        qb = q_ref[:, sl].astype(bf16)
        mx_scr[...] = jnp.full_like(mx_scr, NEG)
        ls_scr[...] = jnp.zeros_like(ls_scr)
        for ki in range(NKV):
            @pl.when(ki * KVB < q_end)
            def _():
                kb = k_ref[ki * KVB:(ki + 1) * KVB, sl].astype(bf16)
                s = jax.lax.dot_general(qb, kb, (((1,), (1,)), ((), ())),
                                        preferred_element_type=f32)
                s = s / jnp.sqrt(jnp.float32(DH))
                row = qi * TB + jax.lax.broadcasted_iota(i32, s.shape, 0)
                col = ki * KVB + jax.lax.broadcasted_iota(i32, s.shape, 1)
                s = jnp.where(col > row, NEG, s)
                s_scr[:, ki * KVB:(ki + 1) * KVB] = s
                mx_scr[:, ki:ki + 1] = jnp.max(s, axis=1, keepdims=True)
        m = jnp.max(mx_scr[...], axis=1, keepdims=True)
        for ki in range(NKV):
            @pl.when(ki * KVB < q_end)
            def _():
                p = jnp.exp(s_scr[:, ki * KVB:(ki + 1) * KVB] - m)
                s_scr[:, ki * KVB:(ki + 1) * KVB] = p
                ls_scr[:, ki:ki + 1] = jnp.sum(p, axis=1, keepdims=True)
        denom = jnp.sum(ls_scr[...], axis=1, keepdims=True)
        for ki in range(NKV):
            @pl.when(ki * KVB < q_end)
            def _():
                att = (s_scr[:, ki * KVB:(ki + 1) * KVB] / denom).astype(bf16)
                vb = v_ref[ki * KVB:(ki + 1) * KVB, sl].astype(bf16)
                acc_scr[:, sl] += jax.lax.dot(att, vb,
                                              preferred_element_type=f32)
    o_ref[...] = acc_scr[...]


def _k2(qkvr):
    return pl.pallas_call(
        _k2_body,
        grid=(H // 2, NT),
        in_specs=[
            pl.BlockSpec((TB, 2 * DH), lambda h, qi: (qi, h)),
            pl.BlockSpec((T, 2 * DH), lambda h, qi: (0, H // 2 + h)),
            pl.BlockSpec((T, 2 * DH), lambda h, qi: (0, H + h)),
        ],
        out_specs=pl.BlockSpec((TB, 2 * DH), lambda h, qi: (qi, h)),
        out_shape=jax.ShapeDtypeStruct((T, D), f32),
        scratch_shapes=[
            pltpu.VMEM((TB, T), f32),
            pltpu.VMEM((TB, 128), f32),
            pltpu.VMEM((TB, 128), f32),
            pltpu.VMEM((TB, 2 * DH), f32),
        ],
    )(qkvr, qkvr, qkvr)


def _k3_body(ctx_ref, x_ref, wo_ref, bo_ref, g2_ref, b2_ref, wg_ref,
             x1_ref, h2_ref, id1_ref, id2_ref, r1_ref, r2_ref,
             p1_ref, p2_ref, cnt_ref, ag_ref):
    i = pl.program_id(0)
    att_out = jax.lax.dot(ctx_ref[...].astype(bf16), wo_ref[...],
                          preferred_element_type=f32) + bo_ref[...]
    x1 = x_ref[...] + att_out
    x1_ref[...] = x1
    h2 = _ln(x1, g2_ref[...], b2_ref[...])
    h2_ref[...] = h2
    s = jax.lax.dot(h2.astype(bf16), wg_ref[...], preferred_element_type=f32)
    lane = jax.lax.broadcasted_iota(i32, s.shape, 1)
    smask = jnp.where(lane < E, s, NEG)
    m1 = jnp.max(smask, axis=1, keepdims=True)
    id1 = jnp.min(jnp.where(smask == m1, lane, 127), axis=1, keepdims=True)
    s2 = jnp.where(lane == id1, NEG, smask)
    m2 = jnp.max(s2, axis=1, keepdims=True)
    id2 = jnp.min(jnp.where(s2 == m2, lane, 127), axis=1, keepdims=True)
    t = jnp.exp(m2 - m1)
    denom = 1.0 + t
    p1_ref[...] = 1.0 / denom
    p2_ref[...] = t / denom
    id1_ref[...] = id1
    id2_ref[...] = id2
    ge = jnp.exp(smask - m1)
    gate = ge / jnp.sum(ge, axis=1, keepdims=True)

    @pl.when(i == 0)
    def _():
        cnt_ref[...] = jnp.zeros_like(cnt_ref)
        ag_ref[...] = jnp.zeros_like(ag_ref)

    carry = cnt_ref[...]
    ag_ref[...] += jnp.sum(gate, axis=0, keepdims=True)
    oh = ((lane == id1) | (lane == id2)).astype(f32)
    rl = jax.lax.broadcasted_iota(i32, (TB, TB), 0)
    cl = jax.lax.broadcasted_iota(i32, (TB, TB), 1)
    ltri = (cl < rl).astype(bf16)
    cum = jax.lax.dot(ltri, oh.astype(bf16), preferred_element_type=f32)
    pos = cum + carry
    r1 = jnp.sum(jnp.where(lane == id1, pos, 0.0), axis=1, keepdims=True)
    r2 = jnp.sum(jnp.where(lane == id2, pos, 0.0), axis=1, keepdims=True)
    r1_ref[...] = r1.astype(i32)
    r2_ref[...] = r2.astype(i32)
    cnt_ref[...] = carry + jnp.sum(oh, axis=0, keepdims=True)


def _k3(ctx, x2d, wo_bf, bo, g2, b2, wg_bf):
    col = lambda i: (i, 0)
    res = lambda i: (0, 0)
    return pl.pallas_call(
        _k3_body,
        grid=(NT,),
        in_specs=[
            pl.BlockSpec((TB, D), col),
            pl.BlockSpec((TB, D), col),
            pl.BlockSpec((D, D), res),
            pl.BlockSpec((1, D), res),
            pl.BlockSpec((1, D), res),
            pl.BlockSpec((1, D), res),
            pl.BlockSpec((D, 128), res),
        ],
        out_specs=[
            pl.BlockSpec((TB, D), col),
            pl.BlockSpec((TB, D), col),
            pl.BlockSpec((TB, 1), col),
            pl.BlockSpec((TB, 1), col),
            pl.BlockSpec((TB, 1), col),
            pl.BlockSpec((TB, 1), col),
            pl.BlockSpec((TB, 1), col),
            pl.BlockSpec((TB, 1), col),
            pl.BlockSpec((1, 128), res),
            pl.BlockSpec((1, 128), res),
        ],
        out_shape=[
            jax.ShapeDtypeStruct((T, D), f32),
            jax.ShapeDtypeStruct((T, D), f32),
            jax.ShapeDtypeStruct((T, 1), i32),
            jax.ShapeDtypeStruct((T, 1), i32),
            jax.ShapeDtypeStruct((T, 1), i32),
            jax.ShapeDtypeStruct((T, 1), i32),
            jax.ShapeDtypeStruct((T, 1), f32),
            jax.ShapeDtypeStruct((T, 1), f32),
            jax.ShapeDtypeStruct((1, 128), f32),
            jax.ShapeDtypeStruct((1, 128), f32),
        ],
    )(ctx, x2d, wo_bf, bo, g2, b2, wg_bf)


def _k3b_body(cnt_ref, ag_ref, id1_ref, id2_ref, r1_ref, r2_ref,
              dA_ref, dB_ref, be_ref, xr_ref, nb_ref, aux_ref):
    cnt = cnt_ref[...]
    lane = jax.lax.broadcasted_iota(i32, (1, 128), 1)
    cnti = cnt.astype(i32)
    blocks_i = (cnti + (BS - 1)) >> 9
    blocks_f = blocks_i.astype(f32)
    padrows_f = (blocks_i << 9).astype(f32)
    pb = jnp.zeros((1, 128), f32)
    cb = blocks_f
    for k in range(1, E):
        pb = pb + jnp.roll(padrows_f, k, axis=1)
        cb = cb + jnp.roll(blocks_f, k, axis=1)
    exb = cb - blocks_f
    nbf = jnp.sum(blocks_f, axis=1, keepdims=True)
    lanef = lane.astype(f32)
    bacc = jnp.zeros((1, 128), i32)
    for e in range(E):
        lo = jnp.sum(jnp.where(lane == e, exb, 0.0), axis=1, keepdims=True)
        hi = jnp.sum(jnp.where(lane == e, cb, 0.0), axis=1, keepdims=True)
        bacc = jnp.where((lanef >= lo) & (lanef < hi), e, bacc)
    be_lastf = jnp.sum(jnp.where(lanef == nbf - 1.0, bacc.astype(f32), 0.0),
                       axis=1, keepdims=True)
    bacc = jnp.where(lanef >= nbf, be_lastf.astype(i32), bacc)
    be_ref[...] = bacc
    xrf = jnp.where(lanef < nbf, lanef, jnp.maximum(nbf - 1.0, 0.0))
    xr_ref[...] = xrf.astype(i32)
    nb_ref[...] = jnp.zeros((1, 128), f32).astype(i32) + nbf.astype(i32)
    aux = (float(E) / float(T * T)) * jnp.sum(cnt * ag_ref[...], axis=1,
                                              keepdims=True)
    aux_ref[...] = jnp.zeros((1, 128), f32) + aux
    lane_big = jax.lax.broadcasted_iota(i32, (T, 128), 1)
    selA = jnp.sum(jnp.where(lane_big == id1_ref[...], pb, 0.0), axis=1,
                   keepdims=True)
    selB = jnp.sum(jnp.where(lane_big == id2_ref[...], pb, 0.0), axis=1,
                   keepdims=True)
    dA_ref[...] = (selA + r1_ref[...].astype(f32)).astype(i32)
    dB_ref[...] = (selB + r2_ref[...].astype(f32)).astype(i32)


def _k3b(cnt128, ag128, id1, id2, r1, r2):
    full = lambda: None
    return pl.pallas_call(
        _k3b_body,
        grid=(1,),
        in_specs=[
            pl.BlockSpec((1, 128), lambda i: (0, 0)),
            pl.BlockSpec((1, 128), lambda i: (0, 0)),
            pl.BlockSpec((T, 1), lambda i: (0, 0)),
            pl.BlockSpec((T, 1), lambda i: (0, 0)),
            pl.BlockSpec((T, 1), lambda i: (0, 0)),
            pl.BlockSpec((T, 1), lambda i: (0, 0)),
        ],
        out_specs=[
            pl.BlockSpec((T, 1), lambda i: (0, 0)),
            pl.BlockSpec((T, 1), lambda i: (0, 0)),
            pl.BlockSpec((1, 128), lambda i: (0, 0)),
            pl.BlockSpec((1, 128), lambda i: (0, 0)),
            pl.BlockSpec((1, 128), lambda i: (0, 0)),
            pl.BlockSpec((1, 128), lambda i: (0, 0)),
        ],
        out_shape=[
            jax.ShapeDtypeStruct((T, 1), i32),
            jax.ShapeDtypeStruct((T, 1), i32),
            jax.ShapeDtypeStruct((1, 128), i32),
            jax.ShapeDtypeStruct((1, 128), i32),
            jax.ShapeDtypeStruct((1, 128), i32),
            jax.ShapeDtypeStruct((1, 128), f32),
        ],
    )(cnt128, ag128, id1, id2, r1, r2)


_SC_MESH = dict(core_axis_name="c", subcore_axis_name="s")


def _sca_body(h2_hbm, dA_hbm, dB_hbm, xpad_hbm,
              rows_v, dA_v, dB_v, semA, semB):
    wid = lax.axis_index("s") * 2 + lax.axis_index("c")
    base = wid * (T // 32)
    pltpu.sync_copy(dA_hbm.at[pl.ds(base, 64)], dA_v)
    pltpu.sync_copy(dB_hbm.at[pl.ds(base, 64)], dB_v)
    pltpu.sync_copy(h2_hbm.at[pl.ds(base, 64)], rows_v)
    cpA = pltpu.async_copy(rows_v, xpad_hbm.at[dA_v], semA)
    cpB = pltpu.async_copy(rows_v, xpad_hbm.at[dB_v], semB)
    cpA.wait()
    cpB.wait()


def _sc_scatter(h2, dAf, dBf):
    mesh = plsc.VectorSubcoreMesh(**_SC_MESH)
    fn = pl.kernel(
        _sca_body, mesh=mesh,
        out_type=[jax.ShapeDtypeStruct((RPAD, D), f32)],
        scratch_types=[
            pltpu.VMEM((64, D), f32),
            pltpu.VMEM((64,), i32), pltpu.VMEM((64,), i32),
            pltpu.SemaphoreType.DMA, pltpu.SemaphoreType.DMA,
        ],
    )
    return fn(h2, dAf, dBf)[0]


def _k4_body(be_s, xr_s, nb_s, x_ref, w1_ref, b1_ref, w2_ref, b2_ref,
             w3_ref, b3_ref, y_ref, xbf_ref):
    i = pl.program_id(0)
    j = pl.program_id(1)

    @pl.when(i < nb_s[0])
    def _():
        @pl.when(j == 0)
        def _():
            xbf_ref[...] = x_ref[...].astype(bf16)

        xb = xbf_ref[...]
        a1 = jax.lax.dot(xb, w1_ref[0].astype(bf16),
                         preferred_element_type=f32) + b1_ref[0]
        a2 = jax.lax.dot(xb, w2_ref[0].astype(bf16),
                         preferred_element_type=f32) + b2_ref[0]
        g = (a1 * jax.nn.sigmoid(a1)) * a2
        part = jax.lax.dot(g.astype(bf16), w3_ref[0].astype(bf16),
                           preferred_element_type=f32)

        @pl.when(j == 0)
        def _():
            y_ref[...] = part

        @pl.when(j > 0)
        def _():
            y_ref[...] = y_ref[...] + part

        @pl.when(j == JB - 1)
        def _():
            y_ref[...] = y_ref[...] + b3_ref[0]


def _k4(be16, xr16, nb16, xpad, w1, b1, w2, b2, w3, b3):
    grid_spec = pltpu.PrefetchScalarGridSpec(
        num_scalar_prefetch=3,
        grid=(NB, JB),
        in_specs=[
            pl.BlockSpec((BS, D), lambda i, j, be, xr, nb: (xr[i], 0)),
            pl.BlockSpec((1, D, FB), lambda i, j, be, xr, nb: (be[i], 0, j)),
            pl.BlockSpec((1, 1, FB), lambda i, j, be, xr, nb: (be[i], 0, j)),
            pl.BlockSpec((1, D, FB), lambda i, j, be, xr, nb: (be[i], 0, j)),
            pl.BlockSpec((1, 1, FB), lambda i, j, be, xr, nb: (be[i], 0, j)),
            pl.BlockSpec((1, FB, D), lambda i, j, be, xr, nb: (be[i], j, 0)),
            pl.BlockSpec((1, 1, D), lambda i, j, be, xr, nb: (be[i], 0, 0)),
        ],
        out_specs=pl.BlockSpec((BS, D), lambda i, j, be, xr, nb: (xr[i], 0)),
        scratch_shapes=[pltpu.VMEM((BS, D), bf16)],
    )
    return pl.pallas_call(
        _k4_body,
        grid_spec=grid_spec,
        out_shape=jax.ShapeDtypeStruct((RPAD, D), f32),
    )(be16, xr16, nb16, xpad, w1, b1.reshape(E, 1, FF), w2,
      b2.reshape(E, 1, FF), w3, b3.reshape(E, 1, D))


def _scb_body(y_hbm, dA_hbm, dB_hbm, yA_hbm, yB_hbm,
              idx_v, buf_v, sem):
    wid = lax.axis_index("s") * 2 + lax.axis_index("c")
    base = wid * (T // 32)
    pltpu.sync_copy(dA_hbm.at[pl.ds(base, 64)], idx_v)
    pltpu.async_copy(y_hbm.at[idx_v], buf_v, sem).wait()
    pltpu.sync_copy(buf_v, yA_hbm.at[pl.ds(base, 64)])
    pltpu.sync_copy(dB_hbm.at[pl.ds(base, 64)], idx_v)
    pltpu.async_copy(y_hbm.at[idx_v], buf_v, sem).wait()
    pltpu.sync_copy(buf_v, yB_hbm.at[pl.ds(base, 64)])


def _sc_gather(y, dA, dB):
    mesh = plsc.VectorSubcoreMesh(**_SC_MESH)
    fn = pl.kernel(
        _scb_body, mesh=mesh,
        out_type=[
            jax.ShapeDtypeStruct((T, D), f32),
            jax.ShapeDtypeStruct((T, D), f32),
        ],
        scratch_types=[
            pltpu.VMEM((64,), i32),
            pltpu.VMEM((64, D), f32),
            pltpu.SemaphoreType.DMA,
        ],
    )
    return fn(y, dA, dB)


def _k5_body(x1_ref, yA_ref, yB_ref, p1_ref, p2_ref, o_ref):
    o_ref[...] = (x1_ref[...] + p1_ref[...] * yA_ref[...]
                  + p2_ref[...] * yB_ref[...])


def _k5(x1, yA, yB, p1, p2):
    col = lambda i: (i, 0)
    return pl.pallas_call(
        _k5_body,
        grid=(NT,),
        in_specs=[
            pl.BlockSpec((TB, D), col),
            pl.BlockSpec((TB, D), col),
            pl.BlockSpec((TB, D), col),
            pl.BlockSpec((TB, 1), col),
            pl.BlockSpec((TB, 1), col),
        ],
        out_specs=pl.BlockSpec((TB, D), col),
        out_shape=jax.ShapeDtypeStruct((T, D), f32),
    )(x1, yA, yB, p1, p2)


def _rope_tables():
    pos = jnp.arange(T, dtype=f32)[:, None]
    cidx = jnp.arange(3 * D)[None, :]
    step = (2 * ((cidx % DH) // 2)).astype(f32)
    inv = jnp.power(10000.0, -2.0 * step / DH)
    ang = pos * inv
    is_qk = cidx < 2 * D
    cosT = jnp.where(is_qk, jnp.cos(ang), 1.0)
    sign = jnp.where(cidx % 2 == 0, -1.0, 1.0)
    sinT = jnp.where(is_qk, jnp.sin(ang) * sign, 0.0)
    return cosT, sinT


def kernel(x, params):
    p = params
    x2d = x.reshape(T, D)
    wqkv_bf = jnp.concatenate([p['Wq'], p['Wk'], p['Wv']], axis=1).astype(bf16)
    cosT, sinT = _rope_tables()
    g1 = p['ln1_scale'].reshape(1, D)
    b1 = p['ln1_shift'].reshape(1, D)
    qkvr = _k1(x2d, wqkv_bf, g1, b1, cosT, sinT)

    ctx = _k2(qkvr)

    wg_pad = jnp.zeros((D, 128), f32).at[:, :E].set(p['Wg']).astype(bf16)
    (x1, h2, id1, id2, r1, r2, p1, p2, cnt128, ag128) = _k3(
        ctx, x2d, p['Wo'].astype(bf16), p['bo'].reshape(1, D),
        p['ln2_scale'].reshape(1, D), p['ln2_shift'].reshape(1, D), wg_pad)

    dA, dB, be128, xr128, nb128, aux128 = _k3b(cnt128, ag128, id1, id2, r1, r2)

    dAf = dA.reshape(T)
    dBf = dB.reshape(T)
    xpad = _sc_scatter(h2, dAf, dBf)

    y = _k4(be128[0, :16], xr128[0, :16], nb128[0, :16], xpad,
            p['w1'], p['b1'], p['w2'], p['b2'], p['w3'], p['b3'])

    yA, yB = _sc_gather(y, dAf, dBf)

    out2d = _k5(x1, yA, yB, p1, p2)
    return (out2d.reshape(B, T, D), aux128[0, 0])


# speedup vs baseline: 1.1960x; 1.1960x over previous
import functools

import jax
import jax.numpy as jnp
from jax import lax
from jax.experimental import pallas as pl
from jax.experimental.pallas import tpu as pltpu
from jax.experimental.pallas import tpu_sc as plsc

B, T, D = 1, 2048, 1024
H, DH = 16, 64
E, K, FF = 8, 2, 4096
EPS = 1e-5

TB = 256
NT = T // TB
BS = 512
NB = 15
RPAD = NB * BS
FB = 1024
JB = FF // FB

NEG = float("-inf")
f32 = jnp.float32
bf16 = jnp.bfloat16
i32 = jnp.int32


def _bdot(a, b):
    return jax.lax.dot(a.astype(bf16), b.astype(bf16),
                       preferred_element_type=f32)


def _ln(xb, g, b):
    m = jnp.mean(xb, axis=1, keepdims=True)
    c = xb - m
    v = jnp.mean(c * c, axis=1, keepdims=True)
    return g * c / jnp.sqrt(v + EPS) + b


def _k1_body(x_ref, w_ref, g_ref, b_ref, cos_ref, sin_ref, o_ref):
    h = _ln(x_ref[...], g_ref[...], b_ref[...])
    qkv = jax.lax.dot(h.astype(bf16), w_ref[...], preferred_element_type=f32)
    lane = jax.lax.broadcasted_iota(i32, qkv.shape, 1)
    r_left = jnp.roll(qkv, -1, axis=1)
    r_right = jnp.roll(qkv, 1, axis=1)
    swapped = jnp.where(lane % 2 == 0, r_left, r_right)
    o_ref[...] = qkv * cos_ref[...] + swapped * sin_ref[...]


def _k1(x2d, wqkv_bf, g1, b1, cosT, sinT):
    return pl.pallas_call(
        _k1_body,
        grid=(NT,),
        in_specs=[
            pl.BlockSpec((TB, D), lambda i: (i, 0)),
            pl.BlockSpec((D, 3 * D), lambda i: (0, 0)),
            pl.BlockSpec((1, D), lambda i: (0, 0)),
            pl.BlockSpec((1, D), lambda i: (0, 0)),
            pl.BlockSpec((TB, 3 * D), lambda i: (i, 0)),
            pl.BlockSpec((TB, 3 * D), lambda i: (i, 0)),
        ],
        out_specs=pl.BlockSpec((TB, 3 * D), lambda i: (i, 0)),
        out_shape=jax.ShapeDtypeStruct((T, 3 * D), f32),
    )(x2d, wqkv_bf, g1, b1, cosT, sinT)


def _k2_body(q_ref, k_ref, v_ref, o_ref):
    qi = pl.program_id(1)
    for hh in range(2):
        sl = slice(hh * DH, (hh + 1) * DH)
        qb = q_ref[:, sl].astype(bf16)
        kb = k_ref[:, sl].astype(bf16)
        s = jax.lax.dot_general(qb, kb, (((1,), (1,)), ((), ())),
                                preferred_element_type=f32)
        s = s / jnp.sqrt(jnp.float32(DH))
        row = qi * TB + jax.lax.broadcasted_iota(i32, s.shape, 0)
        col = jax.lax.broadcasted_iota(i32, s.shape, 1)
        s = jnp.where(col > row, NEG, s)
        m = jnp.max(s, axis=1, keepdims=True)
        p = jnp.exp(s - m)
        att = p / jnp.sum(p, axis=1, keepdims=True)
        o_ref[:, sl] = jax.lax.dot(att.astype(bf16),
                                   v_ref[:, sl].astype(bf16),
                                   preferred_element_type=f32)


def _k2(qkvr):
    return pl.pallas_call(
        _k2_body,
        grid=(H // 2, NT),
        in_specs=[
            pl.BlockSpec((TB, 2 * DH), lambda h, qi: (qi, h)),
            pl.BlockSpec((T, 2 * DH), lambda h, qi: (0, H // 2 + h)),
            pl.BlockSpec((T, 2 * DH), lambda h, qi: (0, H + h)),
        ],
        out_specs=pl.BlockSpec((TB, 2 * DH), lambda h, qi: (qi, h)),
        out_shape=jax.ShapeDtypeStruct((T, D), f32),
    )(qkvr, qkvr, qkvr)


def _k3_body(ctx_ref, x_ref, wo_ref, bo_ref, g2_ref, b2_ref, wg_ref,
             x1_ref, h2_ref, id1_ref, id2_ref, r1_ref, r2_ref,
             p1_ref, p2_ref, cnt_ref, ag_ref):
    i = pl.program_id(0)
    att_out = jax.lax.dot(ctx_ref[...].astype(bf16), wo_ref[...],
                          preferred_element_type=f32) + bo_ref[...]
    x1 = x_ref[...] + att_out
    x1_ref[...] = x1
    h2 = _ln(x1, g2_ref[...], b2_ref[...])
    h2_ref[...] = h2
    s = jax.lax.dot(h2.astype(bf16), wg_ref[...], preferred_element_type=f32)
    lane = jax.lax.broadcasted_iota(i32, s.shape, 1)
    smask = jnp.where(lane < E, s, NEG)
    m1 = jnp.max(smask, axis=1, keepdims=True)
    id1 = jnp.min(jnp.where(smask == m1, lane, 127), axis=1, keepdims=True)
    s2 = jnp.where(lane == id1, NEG, smask)
    m2 = jnp.max(s2, axis=1, keepdims=True)
    id2 = jnp.min(jnp.where(s2 == m2, lane, 127), axis=1, keepdims=True)
    t = jnp.exp(m2 - m1)
    denom = 1.0 + t
    p1_ref[...] = 1.0 / denom
    p2_ref[...] = t / denom
    id1_ref[...] = id1
    id2_ref[...] = id2
    ge = jnp.exp(smask - m1)
    gate = ge / jnp.sum(ge, axis=1, keepdims=True)

    @pl.when(i == 0)
    def _():
        cnt_ref[...] = jnp.zeros_like(cnt_ref)
        ag_ref[...] = jnp.zeros_like(ag_ref)

    carry = cnt_ref[...]
    ag_ref[...] += jnp.sum(gate, axis=0, keepdims=True)
    oh = ((lane == id1) | (lane == id2)).astype(f32)
    rl = jax.lax.broadcasted_iota(i32, (TB, TB), 0)
    cl = jax.lax.broadcasted_iota(i32, (TB, TB), 1)
    ltri = (cl < rl).astype(bf16)
    cum = jax.lax.dot(ltri, oh.astype(bf16), preferred_element_type=f32)
    pos = cum + carry
    r1 = jnp.sum(jnp.where(lane == id1, pos, 0.0), axis=1, keepdims=True)
    r2 = jnp.sum(jnp.where(lane == id2, pos, 0.0), axis=1, keepdims=True)
    r1_ref[...] = r1.astype(i32)
    r2_ref[...] = r2.astype(i32)
    cnt_ref[...] = carry + jnp.sum(oh, axis=0, keepdims=True)


def _k3(ctx, x2d, wo_bf, bo, g2, b2, wg_bf):
    col = lambda i: (i, 0)
    res = lambda i: (0, 0)
    return pl.pallas_call(
        _k3_body,
        grid=(NT,),
        in_specs=[
            pl.BlockSpec((TB, D), col),
            pl.BlockSpec((TB, D), col),
            pl.BlockSpec((D, D), res),
            pl.BlockSpec((1, D), res),
            pl.BlockSpec((1, D), res),
            pl.BlockSpec((1, D), res),
            pl.BlockSpec((D, 128), res),
        ],
        out_specs=[
            pl.BlockSpec((TB, D), col),
            pl.BlockSpec((TB, D), col),
            pl.BlockSpec((TB, 1), col),
            pl.BlockSpec((TB, 1), col),
            pl.BlockSpec((TB, 1), col),
            pl.BlockSpec((TB, 1), col),
            pl.BlockSpec((TB, 1), col),
            pl.BlockSpec((TB, 1), col),
            pl.BlockSpec((1, 128), res),
            pl.BlockSpec((1, 128), res),
        ],
        out_shape=[
            jax.ShapeDtypeStruct((T, D), f32),
            jax.ShapeDtypeStruct((T, D), f32),
            jax.ShapeDtypeStruct((T, 1), i32),
            jax.ShapeDtypeStruct((T, 1), i32),
            jax.ShapeDtypeStruct((T, 1), i32),
            jax.ShapeDtypeStruct((T, 1), i32),
            jax.ShapeDtypeStruct((T, 1), f32),
            jax.ShapeDtypeStruct((T, 1), f32),
            jax.ShapeDtypeStruct((1, 128), f32),
            jax.ShapeDtypeStruct((1, 128), f32),
        ],
    )(ctx, x2d, wo_bf, bo, g2, b2, wg_bf)


def _k3b_body(cnt_ref, ag_ref, id1_ref, id2_ref, r1_ref, r2_ref,
              dA_ref, dB_ref, be_ref, xr_ref, nb_ref, aux_ref):
    cnt = cnt_ref[...]
    lane = jax.lax.broadcasted_iota(i32, (1, 128), 1)
    cnti = cnt.astype(i32)
    blocks_i = (cnti + (BS - 1)) >> 9
    blocks_f = blocks_i.astype(f32)
    padrows_f = (blocks_i << 9).astype(f32)
    pb = jnp.zeros((1, 128), f32)
    cb = blocks_f
    for k in range(1, E):
        pb = pb + jnp.roll(padrows_f, k, axis=1)
        cb = cb + jnp.roll(blocks_f, k, axis=1)
    exb = cb - blocks_f
    nbf = jnp.sum(blocks_f, axis=1, keepdims=True)
    lanef = lane.astype(f32)
    bacc = jnp.zeros((1, 128), i32)
    for e in range(E):
        lo = jnp.sum(jnp.where(lane == e, exb, 0.0), axis=1, keepdims=True)
        hi = jnp.sum(jnp.where(lane == e, cb, 0.0), axis=1, keepdims=True)
        bacc = jnp.where((lanef >= lo) & (lanef < hi), e, bacc)
    be_lastf = jnp.sum(jnp.where(lanef == nbf - 1.0, bacc.astype(f32), 0.0),
                       axis=1, keepdims=True)
    bacc = jnp.where(lanef >= nbf, be_lastf.astype(i32), bacc)
    be_ref[...] = bacc
    xrf = jnp.where(lanef < nbf, lanef, jnp.maximum(nbf - 1.0, 0.0))
    xr_ref[...] = xrf.astype(i32)
    nb_ref[...] = jnp.zeros((1, 128), f32).astype(i32) + nbf.astype(i32)
    aux = (float(E) / float(T * T)) * jnp.sum(cnt * ag_ref[...], axis=1,
                                              keepdims=True)
    aux_ref[...] = jnp.zeros((1, 128), f32) + aux
    lane_big = jax.lax.broadcasted_iota(i32, (T, 128), 1)
    selA = jnp.sum(jnp.where(lane_big == id1_ref[...], pb, 0.0), axis=1,
                   keepdims=True)
    selB = jnp.sum(jnp.where(lane_big == id2_ref[...], pb, 0.0), axis=1,
                   keepdims=True)
    dA_ref[...] = (selA + r1_ref[...].astype(f32)).astype(i32)
    dB_ref[...] = (selB + r2_ref[...].astype(f32)).astype(i32)


def _k3b(cnt128, ag128, id1, id2, r1, r2):
    full = lambda: None
    return pl.pallas_call(
        _k3b_body,
        grid=(1,),
        in_specs=[
            pl.BlockSpec((1, 128), lambda i: (0, 0)),
            pl.BlockSpec((1, 128), lambda i: (0, 0)),
            pl.BlockSpec((T, 1), lambda i: (0, 0)),
            pl.BlockSpec((T, 1), lambda i: (0, 0)),
            pl.BlockSpec((T, 1), lambda i: (0, 0)),
            pl.BlockSpec((T, 1), lambda i: (0, 0)),
        ],
        out_specs=[
            pl.BlockSpec((T, 1), lambda i: (0, 0)),
            pl.BlockSpec((T, 1), lambda i: (0, 0)),
            pl.BlockSpec((1, 128), lambda i: (0, 0)),
            pl.BlockSpec((1, 128), lambda i: (0, 0)),
            pl.BlockSpec((1, 128), lambda i: (0, 0)),
            pl.BlockSpec((1, 128), lambda i: (0, 0)),
        ],
        out_shape=[
            jax.ShapeDtypeStruct((T, 1), i32),
            jax.ShapeDtypeStruct((T, 1), i32),
            jax.ShapeDtypeStruct((1, 128), i32),
            jax.ShapeDtypeStruct((1, 128), i32),
            jax.ShapeDtypeStruct((1, 128), i32),
            jax.ShapeDtypeStruct((1, 128), f32),
        ],
    )(cnt128, ag128, id1, id2, r1, r2)


_SC_MESH = dict(core_axis_name="c", subcore_axis_name="s")


def _sca_body(h2_hbm, dA_hbm, dB_hbm, xpad_hbm,
              rows_v, dA_v, dB_v, semA, semB):
    wid = lax.axis_index("s") * 2 + lax.axis_index("c")
    base = wid * (T // 32)
    pltpu.sync_copy(dA_hbm.at[pl.ds(base, 64)], dA_v)
    pltpu.sync_copy(dB_hbm.at[pl.ds(base, 64)], dB_v)
    pltpu.sync_copy(h2_hbm.at[pl.ds(base, 64)], rows_v)
    cpA = pltpu.async_copy(rows_v, xpad_hbm.at[dA_v], semA)
    cpB = pltpu.async_copy(rows_v, xpad_hbm.at[dB_v], semB)
    cpA.wait()
    cpB.wait()


def _sc_scatter(h2, dAf, dBf):
    mesh = plsc.VectorSubcoreMesh(**_SC_MESH)
    fn = pl.kernel(
        _sca_body, mesh=mesh,
        out_type=[jax.ShapeDtypeStruct((RPAD, D), f32)],
        scratch_types=[
            pltpu.VMEM((64, D), f32),
            pltpu.VMEM((64,), i32), pltpu.VMEM((64,), i32),
            pltpu.SemaphoreType.DMA, pltpu.SemaphoreType.DMA,
        ],
    )
    return fn(h2, dAf, dBf)[0]


def _k4_body(be_s, xr_s, nb_s, x_ref, w1_ref, b1_ref, w2_ref, b2_ref,
             w3_ref, b3_ref, y_ref, xbf_ref):
    i = pl.program_id(0)
    j = pl.program_id(1)

    @pl.when(i < nb_s[0])
    def _():
        @pl.when(j == 0)
        def _():
            xbf_ref[...] = x_ref[...].astype(bf16)

        xb = xbf_ref[...]
        a1 = jax.lax.dot(xb, w1_ref[0].astype(bf16),
                         preferred_element_type=f32) + b1_ref[0]
        a2 = jax.lax.dot(xb, w2_ref[0].astype(bf16),
                         preferred_element_type=f32) + b2_ref[0]
        g = (a1 * jax.nn.sigmoid(a1)) * a2
        part = jax.lax.dot(g.astype(bf16), w3_ref[0].astype(bf16),
                           preferred_element_type=f32)

        @pl.when(j == 0)
        def _():
            y_ref[...] = part

        @pl.when(j > 0)
        def _():
            y_ref[...] = y_ref[...] + part

        @pl.when(j == JB - 1)
        def _():
            y_ref[...] = y_ref[...] + b3_ref[0]


def _k4(be16, xr16, nb16, xpad, w1, b1, w2, b2, w3, b3):
    grid_spec = pltpu.PrefetchScalarGridSpec(
        num_scalar_prefetch=3,
        grid=(NB, JB),
        in_specs=[
            pl.BlockSpec((BS, D), lambda i, j, be, xr, nb: (xr[i], 0)),
            pl.BlockSpec((1, D, FB), lambda i, j, be, xr, nb: (be[i], 0, j)),
            pl.BlockSpec((1, 1, FB), lambda i, j, be, xr, nb: (be[i], 0, j)),
            pl.BlockSpec((1, D, FB), lambda i, j, be, xr, nb: (be[i], 0, j)),
            pl.BlockSpec((1, 1, FB), lambda i, j, be, xr, nb: (be[i], 0, j)),
            pl.BlockSpec((1, FB, D), lambda i, j, be, xr, nb: (be[i], j, 0)),
            pl.BlockSpec((1, 1, D), lambda i, j, be, xr, nb: (be[i], 0, 0)),
        ],
        out_specs=pl.BlockSpec((BS, D), lambda i, j, be, xr, nb: (xr[i], 0)),
        scratch_shapes=[pltpu.VMEM((BS, D), bf16)],
    )
    return pl.pallas_call(
        _k4_body,
        grid_spec=grid_spec,
        out_shape=jax.ShapeDtypeStruct((RPAD, D), f32),
    )(be16, xr16, nb16, xpad, w1, b1.reshape(E, 1, FF), w2,
      b2.reshape(E, 1, FF), w3, b3.reshape(E, 1, D))


def _scb_body(y_hbm, dA_hbm, dB_hbm, yA_hbm, yB_hbm,
              idx_v, buf_v, sem):
    wid = lax.axis_index("s") * 2 + lax.axis_index("c")
    base = wid * (T // 32)
    pltpu.sync_copy(dA_hbm.at[pl.ds(base, 64)], idx_v)
    pltpu.async_copy(y_hbm.at[idx_v], buf_v, sem).wait()
    pltpu.sync_copy(buf_v, yA_hbm.at[pl.ds(base, 64)])
    pltpu.sync_copy(dB_hbm.at[pl.ds(base, 64)], idx_v)
    pltpu.async_copy(y_hbm.at[idx_v], buf_v, sem).wait()
    pltpu.sync_copy(buf_v, yB_hbm.at[pl.ds(base, 64)])


def _sc_gather(y, dA, dB):
    mesh = plsc.VectorSubcoreMesh(**_SC_MESH)
    fn = pl.kernel(
        _scb_body, mesh=mesh,
        out_type=[
            jax.ShapeDtypeStruct((T, D), f32),
            jax.ShapeDtypeStruct((T, D), f32),
        ],
        scratch_types=[
            pltpu.VMEM((64,), i32),
            pltpu.VMEM((64, D), f32),
            pltpu.SemaphoreType.DMA,
        ],
    )
    return fn(y, dA, dB)


def _k5_body(x1_ref, yA_ref, yB_ref, p1_ref, p2_ref, o_ref):
    o_ref[...] = (x1_ref[...] + p1_ref[...] * yA_ref[...]
                  + p2_ref[...] * yB_ref[...])


def _k5(x1, yA, yB, p1, p2):
    col = lambda i: (i, 0)
    return pl.pallas_call(
        _k5_body,
        grid=(NT,),
        in_specs=[
            pl.BlockSpec((TB, D), col),
            pl.BlockSpec((TB, D), col),
            pl.BlockSpec((TB, D), col),
            pl.BlockSpec((TB, 1), col),
            pl.BlockSpec((TB, 1), col),
        ],
        out_specs=pl.BlockSpec((TB, D), col),
        out_shape=jax.ShapeDtypeStruct((T, D), f32),
    )(x1, yA, yB, p1, p2)


def _rope_tables():
    pos = jnp.arange(T, dtype=f32)[:, None]
    cidx = jnp.arange(3 * D)[None, :]
    step = (2 * ((cidx % DH) // 2)).astype(f32)
    inv = jnp.power(10000.0, -2.0 * step / DH)
    ang = pos * inv
    is_qk = cidx < 2 * D
    cosT = jnp.where(is_qk, jnp.cos(ang), 1.0)
    sign = jnp.where(cidx % 2 == 0, -1.0, 1.0)
    sinT = jnp.where(is_qk, jnp.sin(ang) * sign, 0.0)
    return cosT, sinT


def kernel(x, params):
    p = params
    x2d = x.reshape(T, D)
    wqkv_bf = jnp.concatenate([p['Wq'], p['Wk'], p['Wv']], axis=1).astype(bf16)
    cosT, sinT = _rope_tables()
    g1 = p['ln1_scale'].reshape(1, D)
    b1 = p['ln1_shift'].reshape(1, D)
    qkvr = _k1(x2d, wqkv_bf, g1, b1, cosT, sinT)

    ctx = _k2(qkvr)

    wg_pad = jnp.zeros((D, 128), f32).at[:, :E].set(p['Wg']).astype(bf16)
    (x1, h2, id1, id2, r1, r2, p1, p2, cnt128, ag128) = _k3(
        ctx, x2d, p['Wo'].astype(bf16), p['bo'].reshape(1, D),
        p['ln2_scale'].reshape(1, D), p['ln2_shift'].reshape(1, D), wg_pad)

    dA, dB, be128, xr128, nb128, aux128 = _k3b(cnt128, ag128, id1, id2, r1, r2)

    dAf = dA.reshape(T)
    dBf = dB.reshape(T)
    xpad = _sc_scatter(h2, dAf, dBf)

    y = _k4(be128[0, :16], xr128[0, :16], nb128[0, :16], xpad,
            p['w1'], p['b1'], p['w2'], p['b2'], p['w3'], p['b3'])

    yA, yB = _sc_gather(y, dAf, dBf)

    out2d = _k5(x1, yA, yB, p1, p2)
    return (out2d.reshape(B, T, D), aux128[0, 0])


# speedup vs baseline: 1.3587x; 1.1361x over previous
import functools

import jax
import jax.numpy as jnp
from jax import lax
from jax.experimental import pallas as pl
from jax.experimental.pallas import tpu as pltpu
from jax.experimental.pallas import tpu_sc as plsc

B, T, D = 1, 2048, 1024
H, DH = 16, 64
E, K, FF = 8, 2, 4096
EPS = 1e-5

TB = 256
NT = T // TB
BS = 512
NB = 15
RPAD = NB * BS
FB = 1024
JB = FF // FB

NEG = float("-inf")
f32 = jnp.float32
bf16 = jnp.bfloat16
i32 = jnp.int32


def _bdot(a, b):
    return jax.lax.dot(a.astype(bf16), b.astype(bf16),
                       preferred_element_type=f32)


def _ln(xb, g, b):
    m = jnp.mean(xb, axis=1, keepdims=True)
    c = xb - m
    v = jnp.mean(c * c, axis=1, keepdims=True)
    return g * c / jnp.sqrt(v + EPS) + b


def _k1_body(x_ref, wq_ref, wk_ref, wv_ref, g_ref, b_ref, cos_ref, sin_ref,
             o_ref):
    h16 = _ln(x_ref[...], g_ref[...], b_ref[...]).astype(bf16)
    cos = cos_ref[...]
    sin = sin_ref[...]
    lane = jax.lax.broadcasted_iota(i32, (TB, D), 1)
    even = lane % 2 == 0

    def rope(t):
        swapped = jnp.where(even, jnp.roll(t, -1, axis=1),
                            jnp.roll(t, 1, axis=1))
        return t * cos + swapped * sin

    q = jax.lax.dot(h16, wq_ref[...].astype(bf16), preferred_element_type=f32)
    o_ref[:, 0:D] = rope(q).astype(bf16)
    k = jax.lax.dot(h16, wk_ref[...].astype(bf16), preferred_element_type=f32)
    o_ref[:, D:2 * D] = rope(k).astype(bf16)
    v = jax.lax.dot(h16, wv_ref[...].astype(bf16), preferred_element_type=f32)
    o_ref[:, 2 * D:3 * D] = v.astype(bf16)


def _k1(x2d, wq, wk, wv, g1, b1, cosT, sinT):
    res = lambda i: (0, 0)
    return pl.pallas_call(
        _k1_body,
        grid=(NT,),
        in_specs=[
            pl.BlockSpec((TB, D), lambda i: (i, 0)),
            pl.BlockSpec((D, D), res),
            pl.BlockSpec((D, D), res),
            pl.BlockSpec((D, D), res),
            pl.BlockSpec((1, D), res),
            pl.BlockSpec((1, D), res),
            pl.BlockSpec((TB, D), lambda i: (i, 0)),
            pl.BlockSpec((TB, D), lambda i: (i, 0)),
        ],
        out_specs=pl.BlockSpec((TB, 3 * D), lambda i: (i, 0)),
        out_shape=jax.ShapeDtypeStruct((T, 3 * D), bf16),
    )(x2d, wq, wk, wv, g1, b1, cosT, sinT)


def _k2_body(q_ref, k_ref, v_ref, o_ref):
    qi = pl.program_id(1)
    for hh in range(2):
        sl = slice(hh * DH, (hh + 1) * DH)
        qb = q_ref[:, sl]
        kb = k_ref[:, sl]
        s = jax.lax.dot_general(qb, kb, (((1,), (1,)), ((), ())),
                                preferred_element_type=f32)
        s = s / jnp.sqrt(jnp.float32(DH))
        row = qi * TB + jax.lax.broadcasted_iota(i32, s.shape, 0)
        col = jax.lax.broadcasted_iota(i32, s.shape, 1)
        s = jnp.where(col > row, NEG, s)
        m = jnp.max(s, axis=1, keepdims=True)
        p = jnp.exp(s - m)
        att = p / jnp.sum(p, axis=1, keepdims=True)
        o_ref[:, sl] = jax.lax.dot(att.astype(bf16), v_ref[:, sl],
                                   preferred_element_type=f32)


def _k2(qkvr):
    return pl.pallas_call(
        _k2_body,
        grid=(H // 2, NT),
        in_specs=[
            pl.BlockSpec((TB, 2 * DH), lambda h, qi: (qi, h)),
            pl.BlockSpec((T, 2 * DH), lambda h, qi: (0, H // 2 + h)),
            pl.BlockSpec((T, 2 * DH), lambda h, qi: (0, H + h)),
        ],
        out_specs=pl.BlockSpec((TB, 2 * DH), lambda h, qi: (qi, h)),
        out_shape=jax.ShapeDtypeStruct((T, D), f32),
    )(qkvr, qkvr, qkvr)


def _k3_body(ctx_ref, x_ref, wo_ref, bo_ref, g2_ref, b2_ref, wg_ref,
             x1_ref, h2_ref, id1_ref, id2_ref, r1_ref, r2_ref,
             p1_ref, p2_ref, cnt_ref, ag_ref):
    i = pl.program_id(0)
    att_out = jax.lax.dot(ctx_ref[...].astype(bf16), wo_ref[...],
                          preferred_element_type=f32) + bo_ref[...]
    x1 = x_ref[...] + att_out
    x1_ref[...] = x1
    h2 = _ln(x1, g2_ref[...], b2_ref[...])
    h2_ref[...] = h2
    s = jax.lax.dot(h2.astype(bf16), wg_ref[...], preferred_element_type=f32)
    lane = jax.lax.broadcasted_iota(i32, s.shape, 1)
    smask = jnp.where(lane < E, s, NEG)
    m1 = jnp.max(smask, axis=1, keepdims=True)
    id1 = jnp.min(jnp.where(smask == m1, lane, 127), axis=1, keepdims=True)
    s2 = jnp.where(lane == id1, NEG, smask)
    m2 = jnp.max(s2, axis=1, keepdims=True)
    id2 = jnp.min(jnp.where(s2 == m2, lane, 127), axis=1, keepdims=True)
    t = jnp.exp(m2 - m1)
    denom = 1.0 + t
    p1_ref[...] = 1.0 / denom
    p2_ref[...] = t / denom
    id1_ref[...] = id1
    id2_ref[...] = id2
    ge = jnp.exp(smask - m1)
    gate = ge / jnp.sum(ge, axis=1, keepdims=True)

    @pl.when(i == 0)
    def _():
        cnt_ref[...] = jnp.zeros_like(cnt_ref)
        ag_ref[...] = jnp.zeros_like(ag_ref)

    carry = cnt_ref[...]
    ag_ref[...] += jnp.sum(gate, axis=0, keepdims=True)
    oh = ((lane == id1) | (lane == id2)).astype(f32)
    rl = jax.lax.broadcasted_iota(i32, (TB, TB), 0)
    cl = jax.lax.broadcasted_iota(i32, (TB, TB), 1)
    ltri = (cl < rl).astype(bf16)
    cum = jax.lax.dot(ltri, oh.astype(bf16), preferred_element_type=f32)
    pos = cum + carry
    r1 = jnp.sum(jnp.where(lane == id1, pos, 0.0), axis=1, keepdims=True)
    r2 = jnp.sum(jnp.where(lane == id2, pos, 0.0), axis=1, keepdims=True)
    r1_ref[...] = r1.astype(i32)
    r2_ref[...] = r2.astype(i32)
    cnt_ref[...] = carry + jnp.sum(oh, axis=0, keepdims=True)


def _k3(ctx, x2d, wo_bf, bo, g2, b2, wg_bf):
    col = lambda i: (i, 0)
    res = lambda i: (0, 0)
    return pl.pallas_call(
        _k3_body,
        grid=(NT,),
        in_specs=[
            pl.BlockSpec((TB, D), col),
            pl.BlockSpec((TB, D), col),
            pl.BlockSpec((D, D), res),
            pl.BlockSpec((1, D), res),
            pl.BlockSpec((1, D), res),
            pl.BlockSpec((1, D), res),
            pl.BlockSpec((D, 128), res),
        ],
        out_specs=[
            pl.BlockSpec((TB, D), col),
            pl.BlockSpec((TB, D), col),
            pl.BlockSpec((TB, 1), col),
            pl.BlockSpec((TB, 1), col),
            pl.BlockSpec((TB, 1), col),
            pl.BlockSpec((TB, 1), col),
            pl.BlockSpec((TB, 1), col),
            pl.BlockSpec((TB, 1), col),
            pl.BlockSpec((1, 128), res),
            pl.BlockSpec((1, 128), res),
        ],
        out_shape=[
            jax.ShapeDtypeStruct((T, D), f32),
            jax.ShapeDtypeStruct((T, D), f32),
            jax.ShapeDtypeStruct((T, 1), i32),
            jax.ShapeDtypeStruct((T, 1), i32),
            jax.ShapeDtypeStruct((T, 1), i32),
            jax.ShapeDtypeStruct((T, 1), i32),
            jax.ShapeDtypeStruct((T, 1), f32),
            jax.ShapeDtypeStruct((T, 1), f32),
            jax.ShapeDtypeStruct((1, 128), f32),
            jax.ShapeDtypeStruct((1, 128), f32),
        ],
    )(ctx, x2d, wo_bf, bo, g2, b2, wg_bf)


def _k3b_body(cnt_ref, ag_ref, id1_ref, id2_ref, r1_ref, r2_ref,
              dA_ref, dB_ref, be_ref, xr_ref, nb_ref, aux_ref):
    cnt = cnt_ref[...]
    lane = jax.lax.broadcasted_iota(i32, (1, 128), 1)
    cnti = cnt.astype(i32)
    blocks_i = (cnti + (BS - 1)) >> 9
    blocks_f = blocks_i.astype(f32)
    padrows_f = (blocks_i << 9).astype(f32)
    pb = jnp.zeros((1, 128), f32)
    cb = blocks_f
    for k in range(1, E):
        pb = pb + jnp.roll(padrows_f, k, axis=1)
        cb = cb + jnp.roll(blocks_f, k, axis=1)
    exb = cb - blocks_f
    nbf = jnp.sum(blocks_f, axis=1, keepdims=True)
    lanef = lane.astype(f32)
    bacc = jnp.zeros((1, 128), i32)
    for e in range(E):
        lo = jnp.sum(jnp.where(lane == e, exb, 0.0), axis=1, keepdims=True)
        hi = jnp.sum(jnp.where(lane == e, cb, 0.0), axis=1, keepdims=True)
        bacc = jnp.where((lanef >= lo) & (lanef < hi), e, bacc)
    be_lastf = jnp.sum(jnp.where(lanef == nbf - 1.0, bacc.astype(f32), 0.0),
                       axis=1, keepdims=True)
    bacc = jnp.where(lanef >= nbf, be_lastf.astype(i32), bacc)
    be_ref[...] = bacc
    xrf = jnp.where(lanef < nbf, lanef, jnp.maximum(nbf - 1.0, 0.0))
    xr_ref[...] = xrf.astype(i32)
    nb_ref[...] = jnp.zeros((1, 128), f32).astype(i32) + nbf.astype(i32)
    aux = (float(E) / float(T * T)) * jnp.sum(cnt * ag_ref[...], axis=1,
                                              keepdims=True)
    aux_ref[...] = jnp.zeros((1, 128), f32) + aux
    lane_big = jax.lax.broadcasted_iota(i32, (T, 128), 1)
    selA = jnp.sum(jnp.where(lane_big == id1_ref[...], pb, 0.0), axis=1,
                   keepdims=True)
    selB = jnp.sum(jnp.where(lane_big == id2_ref[...], pb, 0.0), axis=1,
                   keepdims=True)
    dA_ref[...] = (selA + r1_ref[...].astype(f32)).astype(i32)
    dB_ref[...] = (selB + r2_ref[...].astype(f32)).astype(i32)


def _k3b(cnt128, ag128, id1, id2, r1, r2):
    full = lambda: None
    return pl.pallas_call(
        _k3b_body,
        grid=(1,),
        in_specs=[
            pl.BlockSpec((1, 128), lambda i: (0, 0)),
            pl.BlockSpec((1, 128), lambda i: (0, 0)),
            pl.BlockSpec((T, 1), lambda i: (0, 0)),
            pl.BlockSpec((T, 1), lambda i: (0, 0)),
            pl.BlockSpec((T, 1), lambda i: (0, 0)),
            pl.BlockSpec((T, 1), lambda i: (0, 0)),
        ],
        out_specs=[
            pl.BlockSpec((T, 1), lambda i: (0, 0)),
            pl.BlockSpec((T, 1), lambda i: (0, 0)),
            pl.BlockSpec((1, 128), lambda i: (0, 0)),
            pl.BlockSpec((1, 128), lambda i: (0, 0)),
            pl.BlockSpec((1, 128), lambda i: (0, 0)),
            pl.BlockSpec((1, 128), lambda i: (0, 0)),
        ],
        out_shape=[
            jax.ShapeDtypeStruct((T, 1), i32),
            jax.ShapeDtypeStruct((T, 1), i32),
            jax.ShapeDtypeStruct((1, 128), i32),
            jax.ShapeDtypeStruct((1, 128), i32),
            jax.ShapeDtypeStruct((1, 128), i32),
            jax.ShapeDtypeStruct((1, 128), f32),
        ],
    )(cnt128, ag128, id1, id2, r1, r2)


_SC_MESH = dict(core_axis_name="c", subcore_axis_name="s")


def _sca_body(h2_hbm, dA_hbm, dB_hbm, xpad_hbm,
              rows_v, dA_v, dB_v, semA, semB):
    wid = lax.axis_index("s") * 2 + lax.axis_index("c")
    base = wid * (T // 32)
    pltpu.sync_copy(dA_hbm.at[pl.ds(base, 64)], dA_v)
    pltpu.sync_copy(dB_hbm.at[pl.ds(base, 64)], dB_v)
    pltpu.sync_copy(h2_hbm.at[pl.ds(base, 64)], rows_v)
    cpA = pltpu.async_copy(rows_v, xpad_hbm.at[dA_v], semA)
    cpB = pltpu.async_copy(rows_v, xpad_hbm.at[dB_v], semB)
    cpA.wait()
    cpB.wait()


def _sc_scatter(h2, dAf, dBf):
    mesh = plsc.VectorSubcoreMesh(**_SC_MESH)
    fn = pl.kernel(
        _sca_body, mesh=mesh,
        out_type=[jax.ShapeDtypeStruct((RPAD, D), f32)],
        scratch_types=[
            pltpu.VMEM((64, D), f32),
            pltpu.VMEM((64,), i32), pltpu.VMEM((64,), i32),
            pltpu.SemaphoreType.DMA, pltpu.SemaphoreType.DMA,
        ],
    )
    return fn(h2, dAf, dBf)[0]


def _k4_body(be_s, xr_s, nb_s, x_ref, w1_ref, b1_ref, w2_ref, b2_ref,
             w3_ref, b3_ref, y_ref, xbf_ref):
    i = pl.program_id(0)
    j = pl.program_id(1)

    @pl.when(i < nb_s[0])
    def _():
        @pl.when(j == 0)
        def _():
            xbf_ref[...] = x_ref[...].astype(bf16)

        xb = xbf_ref[...]
        a1 = jax.lax.dot(xb, w1_ref[0].astype(bf16),
                         preferred_element_type=f32) + b1_ref[0]
        a2 = jax.lax.dot(xb, w2_ref[0].astype(bf16),
                         preferred_element_type=f32) + b2_ref[0]
        g = (a1 * jax.nn.sigmoid(a1)) * a2
        part = jax.lax.dot(g.astype(bf16), w3_ref[0].astype(bf16),
                           preferred_element_type=f32)

        @pl.when(j == 0)
        def _():
            y_ref[...] = part

        @pl.when(j > 0)
        def _():
            y_ref[...] = y_ref[...] + part

        @pl.when(j == JB - 1)
        def _():
            y_ref[...] = y_ref[...] + b3_ref[0]


def _k4(be16, xr16, nb16, xpad, w1, b1, w2, b2, w3, b3):
    grid_spec = pltpu.PrefetchScalarGridSpec(
        num_scalar_prefetch=3,
        grid=(NB, JB),
        in_specs=[
            pl.BlockSpec((BS, D), lambda i, j, be, xr, nb: (xr[i], 0)),
            pl.BlockSpec((1, D, FB), lambda i, j, be, xr, nb: (be[i], 0, j)),
            pl.BlockSpec((1, 1, FB), lambda i, j, be, xr, nb: (be[i], 0, j)),
            pl.BlockSpec((1, D, FB), lambda i, j, be, xr, nb: (be[i], 0, j)),
            pl.BlockSpec((1, 1, FB), lambda i, j, be, xr, nb: (be[i], 0, j)),
            pl.BlockSpec((1, FB, D), lambda i, j, be, xr, nb: (be[i], j, 0)),
            pl.BlockSpec((1, 1, D), lambda i, j, be, xr, nb: (be[i], 0, 0)),
        ],
        out_specs=pl.BlockSpec((BS, D), lambda i, j, be, xr, nb: (xr[i], 0)),
        scratch_shapes=[pltpu.VMEM((BS, D), bf16)],
    )
    return pl.pallas_call(
        _k4_body,
        grid_spec=grid_spec,
        out_shape=jax.ShapeDtypeStruct((RPAD, D), f32),
    )(be16, xr16, nb16, xpad, w1, b1.reshape(E, 1, FF), w2,
      b2.reshape(E, 1, FF), w3, b3.reshape(E, 1, D))


def _scb_body(y_hbm, dA_hbm, dB_hbm, yA_hbm, yB_hbm,
              idx_v, buf_v, sem):
    wid = lax.axis_index("s") * 2 + lax.axis_index("c")
    base = wid * (T // 32)
    pltpu.sync_copy(dA_hbm.at[pl.ds(base, 64)], idx_v)
    pltpu.async_copy(y_hbm.at[idx_v], buf_v, sem).wait()
    pltpu.sync_copy(buf_v, yA_hbm.at[pl.ds(base, 64)])
    pltpu.sync_copy(dB_hbm.at[pl.ds(base, 64)], idx_v)
    pltpu.async_copy(y_hbm.at[idx_v], buf_v, sem).wait()
    pltpu.sync_copy(buf_v, yB_hbm.at[pl.ds(base, 64)])


def _sc_gather(y, dA, dB):
    mesh = plsc.VectorSubcoreMesh(**_SC_MESH)
    fn = pl.kernel(
        _scb_body, mesh=mesh,
        out_type=[
            jax.ShapeDtypeStruct((T, D), f32),
            jax.ShapeDtypeStruct((T, D), f32),
        ],
        scratch_types=[
            pltpu.VMEM((64,), i32),
            pltpu.VMEM((64, D), f32),
            pltpu.SemaphoreType.DMA,
        ],
    )
    return fn(y, dA, dB)


def _k5_body(x1_ref, yA_ref, yB_ref, p1_ref, p2_ref, o_ref):
    o_ref[...] = (x1_ref[...] + p1_ref[...] * yA_ref[...]
                  + p2_ref[...] * yB_ref[...])


def _k5(x1, yA, yB, p1, p2):
    col = lambda i: (i, 0)
    return pl.pallas_call(
        _k5_body,
        grid=(NT,),
        in_specs=[
            pl.BlockSpec((TB, D), col),
            pl.BlockSpec((TB, D), col),
            pl.BlockSpec((TB, D), col),
            pl.BlockSpec((TB, 1), col),
            pl.BlockSpec((TB, 1), col),
        ],
        out_specs=pl.BlockSpec((TB, D), col),
        out_shape=jax.ShapeDtypeStruct((T, D), f32),
    )(x1, yA, yB, p1, p2)


def _rope_tables():
    pos = jnp.arange(T, dtype=f32)[:, None]
    cidx = jnp.arange(D)[None, :]
    step = (2 * ((cidx % DH) // 2)).astype(f32)
    inv = jnp.power(10000.0, -2.0 * step / DH)
    ang = pos * inv
    cosT = jnp.cos(ang)
    sign = jnp.where(cidx % 2 == 0, -1.0, 1.0)
    sinT = jnp.sin(ang) * sign
    return cosT, sinT


def kernel(x, params):
    p = params
    x2d = x.reshape(T, D)
    cosT, sinT = _rope_tables()
    g1 = p['ln1_scale'].reshape(1, D)
    b1 = p['ln1_shift'].reshape(1, D)
    qkvr = _k1(x2d, p['Wq'], p['Wk'], p['Wv'], g1, b1, cosT, sinT)

    ctx = _k2(qkvr)

    wg_pad = jnp.zeros((D, 128), f32).at[:, :E].set(p['Wg']).astype(bf16)
    (x1, h2, id1, id2, r1, r2, p1, p2, cnt128, ag128) = _k3(
        ctx, x2d, p['Wo'].astype(bf16), p['bo'].reshape(1, D),
        p['ln2_scale'].reshape(1, D), p['ln2_shift'].reshape(1, D), wg_pad)

    dA, dB, be128, xr128, nb128, aux128 = _k3b(cnt128, ag128, id1, id2, r1, r2)

    dAf = dA.reshape(T)
    dBf = dB.reshape(T)
    xpad = _sc_scatter(h2, dAf, dBf)

    y = _k4(be128[0, :16], xr128[0, :16], nb128[0, :16], xpad,
            p['w1'], p['b1'], p['w2'], p['b2'], p['w3'], p['b3'])

    yA, yB = _sc_gather(y, dAf, dBf)

    out2d = _k5(x1, yA, yB, p1, p2)
    return (out2d.reshape(B, T, D), aux128[0, 0])


# speedup vs baseline: 1.4177x; 1.0434x over previous
import functools

import jax
import jax.numpy as jnp
from jax import lax
from jax.experimental import pallas as pl
from jax.experimental.pallas import tpu as pltpu
from jax.experimental.pallas import tpu_sc as plsc

B, T, D = 1, 2048, 1024
H, DH = 16, 64
E, K, FF = 8, 2, 4096
EPS = 1e-5

TB = 256
NT = T // TB
BS = 512
NB = 15
RPAD = NB * BS
FB = 1024
JB = FF // FB

NEG = float("-inf")
f32 = jnp.float32
bf16 = jnp.bfloat16
i32 = jnp.int32


def _bdot(a, b):
    return jax.lax.dot(a.astype(bf16), b.astype(bf16),
                       preferred_element_type=f32)


def _ln(xb, g, b):
    m = jnp.mean(xb, axis=1, keepdims=True)
    c = xb - m
    v = jnp.mean(c * c, axis=1, keepdims=True)
    return g * c / jnp.sqrt(v + EPS) + b


def _k1_body(x_ref, wq_ref, wk_ref, wv_ref, g_ref, b_ref, cos_ref, sin_ref,
             o_ref):
    h16 = _ln(x_ref[...], g_ref[...], b_ref[...]).astype(bf16)
    cos = cos_ref[...]
    sin = sin_ref[...]
    lane = jax.lax.broadcasted_iota(i32, (TB, D), 1)
    even = lane % 2 == 0

    def rope(t):
        swapped = jnp.where(even, jnp.roll(t, -1, axis=1),
                            jnp.roll(t, 1, axis=1))
        return t * cos + swapped * sin

    q = jax.lax.dot(h16, wq_ref[...].astype(bf16), preferred_element_type=f32)
    o_ref[:, 0:D] = rope(q).astype(bf16)
    k = jax.lax.dot(h16, wk_ref[...].astype(bf16), preferred_element_type=f32)
    o_ref[:, D:2 * D] = rope(k).astype(bf16)
    v = jax.lax.dot(h16, wv_ref[...].astype(bf16), preferred_element_type=f32)
    o_ref[:, 2 * D:3 * D] = v.astype(bf16)


def _k1(x2d, wq, wk, wv, g1, b1, cosT, sinT):
    res = lambda i: (0, 0)
    return pl.pallas_call(
        _k1_body,
        grid=(NT,),
        in_specs=[
            pl.BlockSpec((TB, D), lambda i: (i, 0)),
            pl.BlockSpec((D, D), res),
            pl.BlockSpec((D, D), res),
            pl.BlockSpec((D, D), res),
            pl.BlockSpec((1, D), res),
            pl.BlockSpec((1, D), res),
            pl.BlockSpec((TB, D), lambda i: (i, 0)),
            pl.BlockSpec((TB, D), lambda i: (i, 0)),
        ],
        out_specs=pl.BlockSpec((TB, 3 * D), lambda i: (i, 0)),
        out_shape=jax.ShapeDtypeStruct((T, 3 * D), bf16),
    )(x2d, wq, wk, wv, g1, b1, cosT, sinT)


def _k2_body(q_lo, q_ref, k_ref, v_ref, o_ref):
    qi = q_lo + pl.program_id(1)
    for hh in range(2):
        sl = slice(hh * DH, (hh + 1) * DH)
        qb = q_ref[:, sl]
        kb = k_ref[:, sl]
        s = jax.lax.dot_general(qb, kb, (((1,), (1,)), ((), ())),
                                preferred_element_type=f32)
        s = s / jnp.sqrt(jnp.float32(DH))
        row = qi * TB + jax.lax.broadcasted_iota(i32, s.shape, 0)
        col = jax.lax.broadcasted_iota(i32, s.shape, 1)
        s = jnp.where(col > row, NEG, s)
        m = jnp.max(s, axis=1, keepdims=True)
        p = jnp.exp(s - m)
        att = p / jnp.sum(p, axis=1, keepdims=True)
        o_ref[:, sl] = jax.lax.dot(att.astype(bf16), v_ref[:, sl],
                                   preferred_element_type=f32).astype(bf16)


def _k2_half(qkvr, kvlen, q_lo, nq):
    import functools as _ft
    return pl.pallas_call(
        _ft.partial(_k2_body, q_lo),
        grid=(H // 2, nq),
        in_specs=[
            pl.BlockSpec((TB, 2 * DH), lambda h, qi: (q_lo + qi, h)),
            pl.BlockSpec((kvlen, 2 * DH), lambda h, qi: (0, H // 2 + h)),
            pl.BlockSpec((kvlen, 2 * DH), lambda h, qi: (0, H + h)),
        ],
        out_specs=pl.BlockSpec((TB, 2 * DH), lambda h, qi: (qi, h)),
        out_shape=jax.ShapeDtypeStruct((nq * TB, D), bf16),
    )(qkvr, qkvr, qkvr)


def _k2(qkvr):
    top = _k2_half(qkvr, T // 2, 0, NT // 2)
    bot = _k2_half(qkvr, T, NT // 2, NT // 2)
    return jnp.concatenate([top, bot], axis=0)


def _k3_body(ctx_ref, x_ref, wo_ref, bo_ref, g2_ref, b2_ref, wg_ref,
             x1_ref, h2_ref, id1_ref, id2_ref, r1_ref, r2_ref,
             p1_ref, p2_ref, cnt_ref, ag_ref):
    i = pl.program_id(0)
    att_out = jax.lax.dot(ctx_ref[...], wo_ref[...],
                          preferred_element_type=f32) + bo_ref[...]
    x1 = x_ref[...] + att_out
    x1_ref[...] = x1
    h2 = _ln(x1, g2_ref[...], b2_ref[...])
    h2_ref[...] = h2
    s = jax.lax.dot(h2.astype(bf16), wg_ref[...], preferred_element_type=f32)
    lane = jax.lax.broadcasted_iota(i32, s.shape, 1)
    smask = jnp.where(lane < E, s, NEG)
    m1 = jnp.max(smask, axis=1, keepdims=True)
    id1 = jnp.min(jnp.where(smask == m1, lane, 127), axis=1, keepdims=True)
    s2 = jnp.where(lane == id1, NEG, smask)
    m2 = jnp.max(s2, axis=1, keepdims=True)
    id2 = jnp.min(jnp.where(s2 == m2, lane, 127), axis=1, keepdims=True)
    t = jnp.exp(m2 - m1)
    denom = 1.0 + t
    p1_ref[...] = 1.0 / denom
    p2_ref[...] = t / denom
    id1_ref[...] = id1
    id2_ref[...] = id2
    ge = jnp.exp(smask - m1)
    gate = ge / jnp.sum(ge, axis=1, keepdims=True)

    @pl.when(i == 0)
    def _():
        cnt_ref[...] = jnp.zeros_like(cnt_ref)
        ag_ref[...] = jnp.zeros_like(ag_ref)

    carry = cnt_ref[...]
    ag_ref[...] += jnp.sum(gate, axis=0, keepdims=True)
    oh = ((lane == id1) | (lane == id2)).astype(f32)
    rl = jax.lax.broadcasted_iota(i32, (TB, TB), 0)
    cl = jax.lax.broadcasted_iota(i32, (TB, TB), 1)
    ltri = (cl < rl).astype(bf16)
    cum = jax.lax.dot(ltri, oh.astype(bf16), preferred_element_type=f32)
    pos = cum + carry
    r1 = jnp.sum(jnp.where(lane == id1, pos, 0.0), axis=1, keepdims=True)
    r2 = jnp.sum(jnp.where(lane == id2, pos, 0.0), axis=1, keepdims=True)
    r1_ref[...] = r1.astype(i32)
    r2_ref[...] = r2.astype(i32)
    cnt_ref[...] = carry + jnp.sum(oh, axis=0, keepdims=True)


def _k3(ctx, x2d, wo_bf, bo, g2, b2, wg_bf):
    col = lambda i: (i, 0)
    res = lambda i: (0, 0)
    return pl.pallas_call(
        _k3_body,
        grid=(NT,),
        in_specs=[
            pl.BlockSpec((TB, D), col),
            pl.BlockSpec((TB, D), col),
            pl.BlockSpec((D, D), res),
            pl.BlockSpec((1, D), res),
            pl.BlockSpec((1, D), res),
            pl.BlockSpec((1, D), res),
            pl.BlockSpec((D, 128), res),
        ],
        out_specs=[
            pl.BlockSpec((TB, D), col),
            pl.BlockSpec((TB, D), col),
            pl.BlockSpec((TB, 1), col),
            pl.BlockSpec((TB, 1), col),
            pl.BlockSpec((TB, 1), col),
            pl.BlockSpec((TB, 1), col),
            pl.BlockSpec((TB, 1), col),
            pl.BlockSpec((TB, 1), col),
            pl.BlockSpec((1, 128), res),
            pl.BlockSpec((1, 128), res),
        ],
        out_shape=[
            jax.ShapeDtypeStruct((T, D), f32),
            jax.ShapeDtypeStruct((T, D), f32),
            jax.ShapeDtypeStruct((T, 1), i32),
            jax.ShapeDtypeStruct((T, 1), i32),
            jax.ShapeDtypeStruct((T, 1), i32),
            jax.ShapeDtypeStruct((T, 1), i32),
            jax.ShapeDtypeStruct((T, 1), f32),
            jax.ShapeDtypeStruct((T, 1), f32),
            jax.ShapeDtypeStruct((1, 128), f32),
            jax.ShapeDtypeStruct((1, 128), f32),
        ],
    )(ctx, x2d, wo_bf, bo, g2, b2, wg_bf)


def _k3b_body(cnt_ref, ag_ref, id1_ref, id2_ref, r1_ref, r2_ref,
              dA_ref, dB_ref, be_ref, xr_ref, nb_ref, aux_ref):
    cnt = cnt_ref[...]
    lane = jax.lax.broadcasted_iota(i32, (1, 128), 1)
    cnti = cnt.astype(i32)
    blocks_i = (cnti + (BS - 1)) >> 9
    blocks_f = blocks_i.astype(f32)
    padrows_f = (blocks_i << 9).astype(f32)
    pb = jnp.zeros((1, 128), f32)
    cb = blocks_f
    for k in range(1, E):
        pb = pb + jnp.roll(padrows_f, k, axis=1)
        cb = cb + jnp.roll(blocks_f, k, axis=1)
    exb = cb - blocks_f
    nbf = jnp.sum(blocks_f, axis=1, keepdims=True)
    lanef = lane.astype(f32)
    bacc = jnp.zeros((1, 128), i32)
    for e in range(E):
        lo = jnp.sum(jnp.where(lane == e, exb, 0.0), axis=1, keepdims=True)
        hi = jnp.sum(jnp.where(lane == e, cb, 0.0), axis=1, keepdims=True)
        bacc = jnp.where((lanef >= lo) & (lanef < hi), e, bacc)
    be_lastf = jnp.sum(jnp.where(lanef == nbf - 1.0, bacc.astype(f32), 0.0),
                       axis=1, keepdims=True)
    bacc = jnp.where(lanef >= nbf, be_lastf.astype(i32), bacc)
    be_ref[...] = bacc
    xrf = jnp.where(lanef < nbf, lanef, jnp.maximum(nbf - 1.0, 0.0))
    xr_ref[...] = xrf.astype(i32)
    nb_ref[...] = jnp.zeros((1, 128), f32).astype(i32) + nbf.astype(i32)
    aux = (float(E) / float(T * T)) * jnp.sum(cnt * ag_ref[...], axis=1,
                                              keepdims=True)
    aux_ref[...] = jnp.zeros((1, 128), f32) + aux
    lane_big = jax.lax.broadcasted_iota(i32, (T, 128), 1)
    selA = jnp.sum(jnp.where(lane_big == id1_ref[...], pb, 0.0), axis=1,
                   keepdims=True)
    selB = jnp.sum(jnp.where(lane_big == id2_ref[...], pb, 0.0), axis=1,
                   keepdims=True)
    dA_ref[...] = (selA + r1_ref[...].astype(f32)).astype(i32)
    dB_ref[...] = (selB + r2_ref[...].astype(f32)).astype(i32)


def _k3b(cnt128, ag128, id1, id2, r1, r2):
    full = lambda: None
    return pl.pallas_call(
        _k3b_body,
        grid=(1,),
        in_specs=[
            pl.BlockSpec((1, 128), lambda i: (0, 0)),
            pl.BlockSpec((1, 128), lambda i: (0, 0)),
            pl.BlockSpec((T, 1), lambda i: (0, 0)),
            pl.BlockSpec((T, 1), lambda i: (0, 0)),
            pl.BlockSpec((T, 1), lambda i: (0, 0)),
            pl.BlockSpec((T, 1), lambda i: (0, 0)),
        ],
        out_specs=[
            pl.BlockSpec((T, 1), lambda i: (0, 0)),
            pl.BlockSpec((T, 1), lambda i: (0, 0)),
            pl.BlockSpec((1, 128), lambda i: (0, 0)),
            pl.BlockSpec((1, 128), lambda i: (0, 0)),
            pl.BlockSpec((1, 128), lambda i: (0, 0)),
            pl.BlockSpec((1, 128), lambda i: (0, 0)),
        ],
        out_shape=[
            jax.ShapeDtypeStruct((T, 1), i32),
            jax.ShapeDtypeStruct((T, 1), i32),
            jax.ShapeDtypeStruct((1, 128), i32),
            jax.ShapeDtypeStruct((1, 128), i32),
            jax.ShapeDtypeStruct((1, 128), i32),
            jax.ShapeDtypeStruct((1, 128), f32),
        ],
    )(cnt128, ag128, id1, id2, r1, r2)


_SC_MESH = dict(core_axis_name="c", subcore_axis_name="s")


def _sca_body(h2_hbm, dA_hbm, dB_hbm, xpad_hbm,
              rows_v, dA_v, dB_v, semA, semB):
    wid = lax.axis_index("s") * 2 + lax.axis_index("c")
    base = wid * (T // 32)
    pltpu.sync_copy(dA_hbm.at[pl.ds(base, 64)], dA_v)
    pltpu.sync_copy(dB_hbm.at[pl.ds(base, 64)], dB_v)
    pltpu.sync_copy(h2_hbm.at[pl.ds(base, 64)], rows_v)
    cpA = pltpu.async_copy(rows_v, xpad_hbm.at[dA_v], semA)
    cpB = pltpu.async_copy(rows_v, xpad_hbm.at[dB_v], semB)
    cpA.wait()
    cpB.wait()


def _sc_scatter(h2, dAf, dBf):
    mesh = plsc.VectorSubcoreMesh(**_SC_MESH)
    fn = pl.kernel(
        _sca_body, mesh=mesh,
        out_type=[jax.ShapeDtypeStruct((RPAD, D), f32)],
        scratch_types=[
            pltpu.VMEM((64, D), f32),
            pltpu.VMEM((64,), i32), pltpu.VMEM((64,), i32),
            pltpu.SemaphoreType.DMA, pltpu.SemaphoreType.DMA,
        ],
    )
    return fn(h2, dAf, dBf)[0]


def _k4_body(be_s, xr_s, nb_s, x_ref, w1_ref, b1_ref, w2_ref, b2_ref,
             w3_ref, b3_ref, y_ref, xbf_ref):
    i = pl.program_id(0)
    j = pl.program_id(1)

    @pl.when(i < nb_s[0])
    def _():
        @pl.when(j == 0)
        def _():
            xbf_ref[...] = x_ref[...].astype(bf16)

        xb = xbf_ref[...]
        a1 = jax.lax.dot(xb, w1_ref[0].astype(bf16),
                         preferred_element_type=f32) + b1_ref[0]
        a2 = jax.lax.dot(xb, w2_ref[0].astype(bf16),
                         preferred_element_type=f32) + b2_ref[0]
        g = (a1 * jax.nn.sigmoid(a1)) * a2
        part = jax.lax.dot(g.astype(bf16), w3_ref[0].astype(bf16),
                           preferred_element_type=f32)

        @pl.when(j == 0)
        def _():
            y_ref[...] = part

        @pl.when(j > 0)
        def _():
            y_ref[...] = y_ref[...] + part

        @pl.when(j == JB - 1)
        def _():
            y_ref[...] = y_ref[...] + b3_ref[0]


def _k4(be16, xr16, nb16, xpad, w1, b1, w2, b2, w3, b3):
    grid_spec = pltpu.PrefetchScalarGridSpec(
        num_scalar_prefetch=3,
        grid=(NB, JB),
        in_specs=[
            pl.BlockSpec((BS, D), lambda i, j, be, xr, nb: (xr[i], 0)),
            pl.BlockSpec((1, D, FB), lambda i, j, be, xr, nb: (be[i], 0, j)),
            pl.BlockSpec((1, 1, FB), lambda i, j, be, xr, nb: (be[i], 0, j)),
            pl.BlockSpec((1, D, FB), lambda i, j, be, xr, nb: (be[i], 0, j)),
            pl.BlockSpec((1, 1, FB), lambda i, j, be, xr, nb: (be[i], 0, j)),
            pl.BlockSpec((1, FB, D), lambda i, j, be, xr, nb: (be[i], j, 0)),
            pl.BlockSpec((1, 1, D), lambda i, j, be, xr, nb: (be[i], 0, 0)),
        ],
        out_specs=pl.BlockSpec((BS, D), lambda i, j, be, xr, nb: (xr[i], 0)),
        scratch_shapes=[pltpu.VMEM((BS, D), bf16)],
    )
    return pl.pallas_call(
        _k4_body,
        grid_spec=grid_spec,
        out_shape=jax.ShapeDtypeStruct((RPAD, D), f32),
    )(be16, xr16, nb16, xpad, w1, b1.reshape(E, 1, FF), w2,
      b2.reshape(E, 1, FF), w3, b3.reshape(E, 1, D))


def _scb_body(y_hbm, dA_hbm, dB_hbm, yA_hbm, yB_hbm,
              idx_v, buf_v, sem):
    wid = lax.axis_index("s") * 2 + lax.axis_index("c")
    base = wid * (T // 32)
    pltpu.sync_copy(dA_hbm.at[pl.ds(base, 64)], idx_v)
    pltpu.async_copy(y_hbm.at[idx_v], buf_v, sem).wait()
    pltpu.sync_copy(buf_v, yA_hbm.at[pl.ds(base, 64)])
    pltpu.sync_copy(dB_hbm.at[pl.ds(base, 64)], idx_v)
    pltpu.async_copy(y_hbm.at[idx_v], buf_v, sem).wait()
    pltpu.sync_copy(buf_v, yB_hbm.at[pl.ds(base, 64)])


def _sc_gather(y, dA, dB):
    mesh = plsc.VectorSubcoreMesh(**_SC_MESH)
    fn = pl.kernel(
        _scb_body, mesh=mesh,
        out_type=[
            jax.ShapeDtypeStruct((T, D), f32),
            jax.ShapeDtypeStruct((T, D), f32),
        ],
        scratch_types=[
            pltpu.VMEM((64,), i32),
            pltpu.VMEM((64, D), f32),
            pltpu.SemaphoreType.DMA,
        ],
    )
    return fn(y, dA, dB)


def _k5_body(x1_ref, yA_ref, yB_ref, p1_ref, p2_ref, o_ref):
    o_ref[...] = (x1_ref[...] + p1_ref[...] * yA_ref[...]
                  + p2_ref[...] * yB_ref[...])


def _k5(x1, yA, yB, p1, p2):
    col = lambda i: (i, 0)
    return pl.pallas_call(
        _k5_body,
        grid=(NT,),
        in_specs=[
            pl.BlockSpec((TB, D), col),
            pl.BlockSpec((TB, D), col),
            pl.BlockSpec((TB, D), col),
            pl.BlockSpec((TB, 1), col),
            pl.BlockSpec((TB, 1), col),
        ],
        out_specs=pl.BlockSpec((TB, D), col),
        out_shape=jax.ShapeDtypeStruct((T, D), f32),
    )(x1, yA, yB, p1, p2)


def _rope_tables():
    pos = jnp.arange(T, dtype=f32)[:, None]
    cidx = jnp.arange(D)[None, :]
    step = (2 * ((cidx % DH) // 2)).astype(f32)
    inv = jnp.power(10000.0, -2.0 * step / DH)
    ang = pos * inv
    cosT = jnp.cos(ang)
    sign = jnp.where(cidx % 2 == 0, -1.0, 1.0)
    sinT = jnp.sin(ang) * sign
    return cosT, sinT


def kernel(x, params):
    p = params
    x2d = x.reshape(T, D)
    cosT, sinT = _rope_tables()
    g1 = p['ln1_scale'].reshape(1, D)
    b1 = p['ln1_shift'].reshape(1, D)
    qkvr = _k1(x2d, p['Wq'], p['Wk'], p['Wv'], g1, b1, cosT, sinT)

    ctx = _k2(qkvr)

    wg_pad = jnp.zeros((D, 128), f32).at[:, :E].set(p['Wg']).astype(bf16)
    (x1, h2, id1, id2, r1, r2, p1, p2, cnt128, ag128) = _k3(
        ctx, x2d, p['Wo'].astype(bf16), p['bo'].reshape(1, D),
        p['ln2_scale'].reshape(1, D), p['ln2_shift'].reshape(1, D), wg_pad)

    dA, dB, be128, xr128, nb128, aux128 = _k3b(cnt128, ag128, id1, id2, r1, r2)

    dAf = dA.reshape(T)
    dBf = dB.reshape(T)
    xpad = _sc_scatter(h2, dAf, dBf)

    y = _k4(be128[0, :16], xr128[0, :16], nb128[0, :16], xpad,
            p['w1'], p['b1'], p['w2'], p['b2'], p['w3'], p['b3'])

    yA, yB = _sc_gather(y, dAf, dBf)

    out2d = _k5(x1, yA, yB, p1, p2)
    return (out2d.reshape(B, T, D), aux128[0, 0])
